# baseline (speedup 1.0000x reference)
import jax
import jax.numpy as jnp
from jax.experimental import pallas as pl
from jax.experimental.pallas import tpu as pltpu

_LEVELS = ((128, 144, 16), (64, 80, 32), (32, 48, 32), (16, 32, 16), (8, 32, 8))
_TOTAL_ROWS = sum(h * wp for h, wp, _ in _LEVELS)


def _rpn_body(x2, x3, x4, x5, x6, wt, wh, bi, bh, out, s2, s3, s4, s5, s6):
    xs_refs = (x2, x3, x4, x5, x6)
    scrs = (s2, s3, s4, s5, s6)
    n_img = x2.shape[0]
    bi_v = bi[0, :][None, :]
    bh_v = bh[0, :][:, None]
    wt_v = wt[...]
    wh_v = wh[...]

    for n in range(n_img):
        for x_ref, scr, (H, Wp, tr) in zip(xs_refs, scrs, _LEVELS):
            W = H
            scr[...] = jnp.zeros((H + 2, Wp, 256), jnp.bfloat16)
            for rb in range(H // tr):
                a = rb * tr
                xt = jnp.transpose(x_ref[n, :, a * W : (a + tr) * W])
                scr[a + 1 : a + 1 + tr, 0:W, :] = xt.reshape(tr, W, 256)

        off = 0
        for scr, (H, Wp, tr) in zip(scrs, _LEVELS):
            M = tr * Wp
            for r in range(H // tr):
                a = r * tr
                xk = [scr[a + ky : a + ky + tr, :, :].reshape(M, 256)
                      for ky in range(3)]
                x3v = jnp.concatenate(xk, axis=1)
                h3 = jnp.dot(x3v, wt_v, preferred_element_type=jnp.float32)
                acc = (jnp.roll(h3[:, 0:256], 1, axis=0)
                       + h3[:, 256:512]
                       + jnp.roll(h3[:, 512:768], -1, axis=0))
                inter = jnp.maximum(acc + bi_v, 0.0).astype(jnp.bfloat16)
                head_t = jax.lax.dot_general(
                    wh_v, inter, (((0,), (1,)), ((), ())),
                    preferred_element_type=jnp.float32) + bh_v
                base = off + r * M
                out[n, :, base : base + M] = head_t
            off += H * Wp


def kernel(p2, p3, p4, p5, p6, image_sizes, annotations,
           W_inter, b_inter, W_logit, b_logit, W_reg, b_reg):
    del image_sizes, annotations
    n = p2.shape[0]
    feats = tuple(
        x.astype(jnp.bfloat16).reshape(n, 256, -1)
        for x in (p2, p3, p4, p5, p6))
    wt = jnp.transpose(W_inter, (2, 1, 3, 0)).reshape(768, 768)
    wt = wt.astype(jnp.bfloat16)
    wh = jnp.concatenate([W_logit[:, :, 0, 0].T, W_reg[:, :, 0, 0].T], axis=1)
    wh = wh.astype(jnp.bfloat16)
    bi = b_inter.reshape(1, 256).astype(jnp.float32)
    bh = jnp.concatenate([b_logit, b_reg]).reshape(1, 15).astype(jnp.float32)

    in_specs = [pl.BlockSpec(x.shape, lambda: (0, 0, 0)) for x in feats]
    in_specs += [
        pl.BlockSpec((768, 768), lambda: (0, 0)),
        pl.BlockSpec((256, 15), lambda: (0, 0)),
        pl.BlockSpec((1, 256), lambda: (0, 0)),
        pl.BlockSpec((1, 15), lambda: (0, 0)),
    ]
    out = pl.pallas_call(
        _rpn_body,
        in_specs=in_specs,
        out_specs=pl.BlockSpec((n, 15, _TOTAL_ROWS), lambda: (0, 0, 0)),
        out_shape=jax.ShapeDtypeStruct((n, 15, _TOTAL_ROWS), jnp.float32),
        scratch_shapes=[
            pltpu.VMEM((h + 2, wp, 256), jnp.bfloat16) for h, wp, _ in _LEVELS
        ],
    )(*feats, wt, wh, bi, bh)

    outp = jnp.transpose(out, (0, 2, 1))
    segs = []
    off = 0
    for H, Wp, _ in _LEVELS:
        seg = outp[:, off : off + H * Wp, :].reshape(n, H, Wp, 15)[:, :, :H, :]
        segs.append(seg.reshape(n, H * H, 15))
        off += H * Wp
    full = jnp.concatenate(segs, axis=1)
    tot = full.shape[1]
    logits = full[:, :, :3].reshape(n, tot * 3)
    deltas = full[:, :, 3:].reshape(n, tot * 3, 4)
    return (logits, deltas)
